# Initial kernel scaffold; baseline (speedup 1.0000x reference)
import jax
import jax.numpy as jnp
from jax.experimental import pallas as pl


def kernel(x, edge_index, W1, b1, W2, b2, fc1_W, fc1_b, fc2_W, fc2_b):
    raise NotImplementedError("write your pallas kernel here")


# speedup vs baseline: 10.6089x; 10.6089x over previous
import functools

import jax
import jax.numpy as jnp
from jax import lax
from jax.experimental import pallas as pl
from jax.experimental.pallas import tpu as pltpu
from jax.experimental.pallas import tpu_sc as plsc

N = 10000
E = 320000
NC = 2
NS = 16
NW = NC * NS
CHUNK = 128
NCH = 80
EPW = NCH * CHUNK
EPAD = NW * EPW
NPAD = 10240
SL = NPAD // NS
D = 64

_mesh = plsc.VectorSubcoreMesh(core_axis_name="c", subcore_axis_name="s")


@functools.partial(
    pl.kernel, mesh=_mesh,
    compiler_params=pltpu.CompilerParams(use_tc_tiling_on_sc=False),
    out_type=jax.ShapeDtypeStruct((NC, NPAD), jnp.float32),
    scratch_types=[
        pltpu.VMEM((NCH, CHUNK), jnp.int32),
        pltpu.VMEM((CHUNK,), jnp.float32),
        pltpu.VMEM_SHARED((NPAD,), jnp.float32),
    ],
)
def _deg_kernel(dst_hbm, ones_hbm, zeros_hbm, out_hbm, idx_v, ones_v, acc_sh):
    cid = lax.axis_index("c")
    sid = lax.axis_index("s")
    wid = sid * NC + cid
    pltpu.sync_copy(zeros_hbm.at[pl.ds(sid * SL, SL)], acc_sh.at[pl.ds(sid * SL, SL)])
    pltpu.sync_copy(ones_hbm, ones_v)
    pltpu.sync_copy(dst_hbm.at[wid], idx_v)
    plsc.subcore_barrier()

    def body(j, carry):
        pltpu.sync_copy(ones_v, acc_sh.at[idx_v.at[j]], add=True)
        return carry

    lax.fori_loop(0, NCH, body, 0)
    plsc.subcore_barrier()
    pltpu.sync_copy(acc_sh.at[pl.ds(sid * SL, SL)], out_hbm.at[cid, pl.ds(sid * SL, SL)])


def _make_scatter(num_tables):
    out_struct = jax.ShapeDtypeStruct((NC, NPAD, D), jnp.float32)

    @functools.partial(
        pl.kernel, mesh=_mesh,
        compiler_params=pltpu.CompilerParams(use_tc_tiling_on_sc=False),
        out_type=[out_struct] * num_tables,
        scratch_types=[
            pltpu.VMEM((NCH, CHUNK), jnp.int32),
            pltpu.VMEM((NCH, CHUNK), jnp.int32),
            pltpu.VMEM((CHUNK, D), jnp.float32),
            pltpu.VMEM((CHUNK, D), jnp.float32),
            pltpu.VMEM_SHARED((NPAD, D), jnp.float32),
            pltpu.SemaphoreType.DMA,
            pltpu.SemaphoreType.DMA,
        ],
    )
    def _scatter_kernel(*args):
        tables = args[:num_tables]
        src_hbm, dst_hbm, zeros_hbm = args[num_tables:num_tables + 3]
        outs = args[num_tables + 3:2 * num_tables + 3]
        srcv, dstv, rows0, rows1, acc_sh, sem0, sem1 = args[2 * num_tables + 3:]
        cid = lax.axis_index("c")
        sid = lax.axis_index("s")
        wid = sid * NC + cid
        sl = pl.ds(sid * SL, SL)
        pltpu.sync_copy(src_hbm.at[wid], srcv)
        pltpu.sync_copy(dst_hbm.at[wid], dstv)
        for g_hbm, out_hbm in zip(tables, outs):
            pltpu.sync_copy(zeros_hbm.at[sl], acc_sh.at[sl])
            plsc.subcore_barrier()

            def body(j2, carry):
                j = 2 * j2
                a = pltpu.async_copy(g_hbm.at[srcv.at[j]], rows0, sem0)
                b = pltpu.async_copy(g_hbm.at[srcv.at[j + 1]], rows1, sem1)
                a.wait()
                pltpu.sync_copy(rows0, acc_sh.at[dstv.at[j]], add=True)
                b.wait()
                pltpu.sync_copy(rows1, acc_sh.at[dstv.at[j + 1]], add=True)
                return carry

            lax.fori_loop(0, NCH // 2, body, 0)
            plsc.subcore_barrier()
            pltpu.sync_copy(acc_sh.at[sl], out_hbm.at[cid, sl])

    return _scatter_kernel


_scatter_dual = _make_scatter(2)
_scatter_single = _make_scatter(1)


BN = 1000


def _mm_scale_body(x_ref, w_ref, d0_ref, d1_ref, outa_ref, outb_ref):
    dinv = lax.rsqrt(d0_ref[...] + d1_ref[...] + 1.0)
    h = jnp.dot(x_ref[...], w_ref[...], preferred_element_type=jnp.float32)
    g = h * dinv
    outa_ref[...] = g[:, :D]
    outb_ref[...] = g[:, D:]


def _mm_scale(x, W, d0, d1):
    return pl.pallas_call(
        _mm_scale_body,
        grid=(N // BN,),
        in_specs=[
            pl.BlockSpec((BN, 128), lambda i: (i, 0)),
            pl.BlockSpec((128, 128), lambda i: (0, 0)),
            pl.BlockSpec((BN, 1), lambda i: (i, 0)),
            pl.BlockSpec((BN, 1), lambda i: (i, 0)),
        ],
        out_specs=[
            pl.BlockSpec((BN, D), lambda i: (i, 0)),
            pl.BlockSpec((BN, D), lambda i: (i, 0)),
        ],
        out_shape=[
            jax.ShapeDtypeStruct((N, D), jnp.float32),
            jax.ShapeDtypeStruct((N, D), jnp.float32),
        ],
    )(x, W, d0, d1)


def _layer2_body(aa0_ref, aa1_ref, ab0_ref, ab1_ref, ga_ref, gb_ref,
                 d0_ref, d1_ref, b1_ref, w2_ref, out_ref):
    dinv = lax.rsqrt(d0_ref[...] + d1_ref[...] + 1.0)
    b1 = b1_ref[...]
    h1a = jnp.maximum(dinv * (aa0_ref[...] + aa1_ref[...] + ga_ref[...]) + b1[:, :D], 0.0)
    h1b = jnp.maximum(dinv * (ab0_ref[...] + ab1_ref[...] + gb_ref[...]) + b1[:, D:], 0.0)
    w2 = w2_ref[...]
    h2 = (jnp.dot(h1a, w2[:D], preferred_element_type=jnp.float32)
          + jnp.dot(h1b, w2[D:], preferred_element_type=jnp.float32))
    out_ref[...] = dinv * h2


def _layer2(aa0, aa1, ab0, ab1, ga, gb, d0, d1, b1r, W2):
    return pl.pallas_call(
        _layer2_body,
        grid=(N // BN,),
        in_specs=[
            pl.BlockSpec((BN, D), lambda i: (i, 0)),
            pl.BlockSpec((BN, D), lambda i: (i, 0)),
            pl.BlockSpec((BN, D), lambda i: (i, 0)),
            pl.BlockSpec((BN, D), lambda i: (i, 0)),
            pl.BlockSpec((BN, D), lambda i: (i, 0)),
            pl.BlockSpec((BN, D), lambda i: (i, 0)),
            pl.BlockSpec((BN, 1), lambda i: (i, 0)),
            pl.BlockSpec((BN, 1), lambda i: (i, 0)),
            pl.BlockSpec((1, 128), lambda i: (0, 0)),
            pl.BlockSpec((128, 64), lambda i: (0, 0)),
        ],
        out_specs=pl.BlockSpec((BN, 64), lambda i: (i, 0)),
        out_shape=jax.ShapeDtypeStruct((N, 64), jnp.float32),
    )(aa0, aa1, ab0, ab1, ga, gb, d0, d1, b1r, W2)


_M = float(N * 2)


def _head_body(a0_ref, a1_ref, g2_ref, d0_ref, d1_ref, b2_ref,
               fc1w_ref, fc1b_ref, fc2w_ref, fc2b_ref,
               assign_ref, var_ref, acc_ref):
    i = pl.program_id(0)

    @pl.when(i == 0)
    def _():
        acc_ref[0] = 0.0
        acc_ref[1] = 0.0

    dinv = lax.rsqrt(d0_ref[...] + d1_ref[...] + 1.0)
    h2 = dinv * (a0_ref[...] + a1_ref[...] + g2_ref[...]) + b2_ref[...]
    a1 = jnp.tanh(jnp.dot(h2, fc1w_ref[...], preferred_element_type=jnp.float32)
                  + fc1b_ref[...])
    logits = jnp.dot(a1, fc2w_ref[...], preferred_element_type=jnp.float32) + fc2b_ref[...]
    m = jnp.max(logits, axis=1, keepdims=True)
    e = jnp.exp(logits - m)
    assign = e / jnp.sum(e, axis=1, keepdims=True)
    assign_ref[...] = assign
    c = assign - 0.5
    acc_ref[0] += jnp.sum(c)
    acc_ref[1] += jnp.sum(c * c)

    @pl.when(i == pl.num_programs(0) - 1)
    def _():
        s = acc_ref[0]
        q = acc_ref[1]
        v = (q - s * s / _M) / (_M - 1.0)
        var_ref[...] = jnp.broadcast_to(v, (1, 1))


def _head(a0, a1, g2, d0, d1, b2r, fc1_W, fc1_br, fc2_W, fc2_br):
    return pl.pallas_call(
        _head_body,
        grid=(N // BN,),
        in_specs=[
            pl.BlockSpec((BN, 64), lambda i: (i, 0)),
            pl.BlockSpec((BN, 64), lambda i: (i, 0)),
            pl.BlockSpec((BN, 64), lambda i: (i, 0)),
            pl.BlockSpec((BN, 1), lambda i: (i, 0)),
            pl.BlockSpec((BN, 1), lambda i: (i, 0)),
            pl.BlockSpec((1, 64), lambda i: (0, 0)),
            pl.BlockSpec((64, 32), lambda i: (0, 0)),
            pl.BlockSpec((1, 32), lambda i: (0, 0)),
            pl.BlockSpec((32, 2), lambda i: (0, 0)),
            pl.BlockSpec((1, 2), lambda i: (0, 0)),
        ],
        out_specs=[
            pl.BlockSpec((BN, 2), lambda i: (i, 0)),
            pl.BlockSpec((1, 1), lambda i: (0, 0)),
        ],
        out_shape=[
            jax.ShapeDtypeStruct((N, 2), jnp.float32),
            jax.ShapeDtypeStruct((1, 1), jnp.float32),
        ],
        scratch_shapes=[pltpu.SMEM((2,), jnp.float32)],
    )(a0, a1, g2, d0, d1, b2r, fc1_W, fc1_br, fc2_W, fc2_br)


def kernel(x, edge_index, W1, b1, W2, b2, fc1_W, fc1_b, fc2_W, fc2_b):
    pad = EPAD - E
    src_p = jnp.concatenate(
        [edge_index[0], jnp.zeros((pad,), jnp.int32)]).reshape(NW, NCH, CHUNK)
    dst_p = jnp.concatenate(
        [edge_index[1], jnp.full((pad,), N, jnp.int32)]).reshape(NW, NCH, CHUNK)

    ones_c = jnp.ones((CHUNK,), jnp.float32)
    zeros_d = jnp.zeros((NPAD,), jnp.float32)
    zeros_64 = jnp.zeros((NPAD, D), jnp.float32)

    deg_parts = _deg_kernel(dst_p, ones_c, zeros_d)
    d0 = deg_parts[0, :N, None]
    d1 = deg_parts[1, :N, None]

    g1a, g1b = _mm_scale(x, W1, d0, d1)
    acc_a, acc_b = _scatter_dual(g1a, g1b, src_p, dst_p, zeros_64)
    g2 = _layer2(acc_a[0, :N], acc_a[1, :N], acc_b[0, :N], acc_b[1, :N],
                 g1a, g1b, d0, d1, b1[None, :], W2)
    acc2, = _scatter_single(g2, src_p, dst_p, zeros_64)
    assign, var = _head(acc2[0, :N], acc2[1, :N], g2, d0, d1, b2[None, :],
                        fc1_W, fc1_b[None, :], fc2_W, fc2_b[None, :])
    return assign, var[0, 0]


# speedup vs baseline: 11.6092x; 1.0943x over previous
import functools

import jax
import jax.numpy as jnp
from jax import lax
from jax.experimental import pallas as pl
from jax.experimental.pallas import tpu as pltpu
from jax.experimental.pallas import tpu_sc as plsc

N = 10000
E = 320000
NC = 2
NS = 16
NW = NC * NS
CHUNK = 128
NCH = 80
EPW = NCH * CHUNK
EPAD = NW * EPW
NPAD = 10240
SL = NPAD // NS
D = 64

_mesh = plsc.VectorSubcoreMesh(core_axis_name="c", subcore_axis_name="s")


@functools.partial(
    pl.kernel, mesh=_mesh,
    compiler_params=pltpu.CompilerParams(use_tc_tiling_on_sc=False),
    out_type=jax.ShapeDtypeStruct((NC, NPAD), jnp.float32),
    scratch_types=[
        pltpu.VMEM((NCH, CHUNK), jnp.int32),
        pltpu.VMEM((CHUNK,), jnp.float32),
        pltpu.VMEM_SHARED((NPAD,), jnp.float32),
    ],
)
def _deg_kernel(dst_hbm, ones_hbm, zeros_hbm, out_hbm, idx_v, ones_v, acc_sh):
    cid = lax.axis_index("c")
    sid = lax.axis_index("s")
    wid = sid * NC + cid
    pltpu.sync_copy(zeros_hbm.at[pl.ds(sid * SL, SL)], acc_sh.at[pl.ds(sid * SL, SL)])
    pltpu.sync_copy(ones_hbm, ones_v)
    pltpu.sync_copy(dst_hbm.at[wid], idx_v)
    plsc.subcore_barrier()

    def body(j, carry):
        pltpu.sync_copy(ones_v, acc_sh.at[idx_v.at[j]], add=True)
        return carry

    lax.fori_loop(0, NCH, body, 0)
    plsc.subcore_barrier()
    pltpu.sync_copy(acc_sh.at[pl.ds(sid * SL, SL)], out_hbm.at[cid, pl.ds(sid * SL, SL)])


U = 4


def _make_scatter(num_tables):
    out_struct = jax.ShapeDtypeStruct((NC, NPAD, D), jnp.float32)

    @functools.partial(
        pl.kernel, mesh=_mesh,
        compiler_params=pltpu.CompilerParams(use_tc_tiling_on_sc=False),
        out_type=[out_struct] * num_tables,
        scratch_types=[
            pltpu.VMEM((NCH, CHUNK), jnp.int32),
            pltpu.VMEM((NCH, CHUNK), jnp.int32),
        ] + [pltpu.VMEM((CHUNK, D), jnp.float32)] * U
          + [pltpu.VMEM_SHARED((NPAD, D), jnp.float32)]
          + [pltpu.SemaphoreType.DMA] * (2 * U),
    )
    def _scatter_kernel(*args):
        tables = args[:num_tables]
        src_hbm, dst_hbm, zeros_hbm = args[num_tables:num_tables + 3]
        outs = args[num_tables + 3:2 * num_tables + 3]
        rest = args[2 * num_tables + 3:]
        srcv, dstv = rest[0], rest[1]
        rows = rest[2:2 + U]
        acc_sh = rest[2 + U]
        gsem = rest[3 + U:3 + 2 * U]
        ssem = rest[3 + 2 * U:3 + 3 * U]
        cid = lax.axis_index("c")
        sid = lax.axis_index("s")
        wid = sid * NC + cid
        sl = pl.ds(sid * SL, SL)
        pltpu.sync_copy(src_hbm.at[wid], srcv)
        pltpu.sync_copy(dst_hbm.at[wid], dstv)
        for g_hbm, out_hbm in zip(tables, outs):
            pltpu.sync_copy(zeros_hbm.at[sl], acc_sh.at[sl])
            plsc.subcore_barrier()

            for b in range(U):
                pltpu.async_copy(g_hbm.at[srcv.at[b]], rows[b], gsem[b])

            def round_body(r, carry):
                base = U * r
                for b in range(U):
                    pltpu.make_async_copy(
                        g_hbm.at[srcv.at[base + b]], rows[b], gsem[b]).wait()
                    pltpu.async_copy(
                        rows[b], acc_sh.at[dstv.at[base + b]], ssem[b], add=True)
                for b in range(U):
                    pltpu.make_async_copy(
                        rows[b], acc_sh.at[dstv.at[base + b]], ssem[b]).wait()
                    pltpu.async_copy(
                        g_hbm.at[srcv.at[base + U + b]], rows[b], gsem[b])
                return carry

            lax.fori_loop(0, NCH // U - 1, round_body, 0)

            base = NCH - U
            for b in range(U):
                pltpu.make_async_copy(
                    g_hbm.at[srcv.at[base + b]], rows[b], gsem[b]).wait()
                pltpu.async_copy(
                    rows[b], acc_sh.at[dstv.at[base + b]], ssem[b], add=True)
            for b in range(U):
                pltpu.make_async_copy(
                    rows[b], acc_sh.at[dstv.at[base + b]], ssem[b]).wait()

            plsc.subcore_barrier()
            pltpu.sync_copy(acc_sh.at[sl], out_hbm.at[cid, sl])

    return _scatter_kernel


_scatter_dual = _make_scatter(2)
_scatter_single = _make_scatter(1)


BN = 1000


def _mm_scale_body(x_ref, w_ref, d0_ref, d1_ref, outa_ref, outb_ref):
    dinv = lax.rsqrt(d0_ref[...] + d1_ref[...] + 1.0)
    h = jnp.dot(x_ref[...], w_ref[...], preferred_element_type=jnp.float32)
    g = h * dinv
    outa_ref[...] = g[:, :D]
    outb_ref[...] = g[:, D:]


def _mm_scale(x, W, d0, d1):
    return pl.pallas_call(
        _mm_scale_body,
        grid=(N // BN,),
        in_specs=[
            pl.BlockSpec((BN, 128), lambda i: (i, 0)),
            pl.BlockSpec((128, 128), lambda i: (0, 0)),
            pl.BlockSpec((BN, 1), lambda i: (i, 0)),
            pl.BlockSpec((BN, 1), lambda i: (i, 0)),
        ],
        out_specs=[
            pl.BlockSpec((BN, D), lambda i: (i, 0)),
            pl.BlockSpec((BN, D), lambda i: (i, 0)),
        ],
        out_shape=[
            jax.ShapeDtypeStruct((N, D), jnp.float32),
            jax.ShapeDtypeStruct((N, D), jnp.float32),
        ],
    )(x, W, d0, d1)


def _layer2_body(aa0_ref, aa1_ref, ab0_ref, ab1_ref, ga_ref, gb_ref,
                 d0_ref, d1_ref, b1_ref, w2_ref, out_ref):
    dinv = lax.rsqrt(d0_ref[...] + d1_ref[...] + 1.0)
    b1 = b1_ref[...]
    h1a = jnp.maximum(dinv * (aa0_ref[...] + aa1_ref[...] + ga_ref[...]) + b1[:, :D], 0.0)
    h1b = jnp.maximum(dinv * (ab0_ref[...] + ab1_ref[...] + gb_ref[...]) + b1[:, D:], 0.0)
    w2 = w2_ref[...]
    h2 = (jnp.dot(h1a, w2[:D], preferred_element_type=jnp.float32)
          + jnp.dot(h1b, w2[D:], preferred_element_type=jnp.float32))
    out_ref[...] = dinv * h2


def _layer2(aa0, aa1, ab0, ab1, ga, gb, d0, d1, b1r, W2):
    return pl.pallas_call(
        _layer2_body,
        grid=(N // BN,),
        in_specs=[
            pl.BlockSpec((BN, D), lambda i: (i, 0)),
            pl.BlockSpec((BN, D), lambda i: (i, 0)),
            pl.BlockSpec((BN, D), lambda i: (i, 0)),
            pl.BlockSpec((BN, D), lambda i: (i, 0)),
            pl.BlockSpec((BN, D), lambda i: (i, 0)),
            pl.BlockSpec((BN, D), lambda i: (i, 0)),
            pl.BlockSpec((BN, 1), lambda i: (i, 0)),
            pl.BlockSpec((BN, 1), lambda i: (i, 0)),
            pl.BlockSpec((1, 128), lambda i: (0, 0)),
            pl.BlockSpec((128, 64), lambda i: (0, 0)),
        ],
        out_specs=pl.BlockSpec((BN, 64), lambda i: (i, 0)),
        out_shape=jax.ShapeDtypeStruct((N, 64), jnp.float32),
    )(aa0, aa1, ab0, ab1, ga, gb, d0, d1, b1r, W2)


_M = float(N * 2)


def _head_body(a0_ref, a1_ref, g2_ref, d0_ref, d1_ref, b2_ref,
               fc1w_ref, fc1b_ref, fc2w_ref, fc2b_ref,
               assign_ref, var_ref, acc_ref):
    i = pl.program_id(0)

    @pl.when(i == 0)
    def _():
        acc_ref[0] = 0.0
        acc_ref[1] = 0.0

    dinv = lax.rsqrt(d0_ref[...] + d1_ref[...] + 1.0)
    h2 = dinv * (a0_ref[...] + a1_ref[...] + g2_ref[...]) + b2_ref[...]
    a1 = jnp.tanh(jnp.dot(h2, fc1w_ref[...], preferred_element_type=jnp.float32)
                  + fc1b_ref[...])
    logits = jnp.dot(a1, fc2w_ref[...], preferred_element_type=jnp.float32) + fc2b_ref[...]
    m = jnp.max(logits, axis=1, keepdims=True)
    e = jnp.exp(logits - m)
    assign = e / jnp.sum(e, axis=1, keepdims=True)
    assign_ref[...] = assign
    c = assign - 0.5
    acc_ref[0] += jnp.sum(c)
    acc_ref[1] += jnp.sum(c * c)

    @pl.when(i == pl.num_programs(0) - 1)
    def _():
        s = acc_ref[0]
        q = acc_ref[1]
        v = (q - s * s / _M) / (_M - 1.0)
        var_ref[...] = jnp.broadcast_to(v, (1, 1))


def _head(a0, a1, g2, d0, d1, b2r, fc1_W, fc1_br, fc2_W, fc2_br):
    return pl.pallas_call(
        _head_body,
        grid=(N // BN,),
        in_specs=[
            pl.BlockSpec((BN, 64), lambda i: (i, 0)),
            pl.BlockSpec((BN, 64), lambda i: (i, 0)),
            pl.BlockSpec((BN, 64), lambda i: (i, 0)),
            pl.BlockSpec((BN, 1), lambda i: (i, 0)),
            pl.BlockSpec((BN, 1), lambda i: (i, 0)),
            pl.BlockSpec((1, 64), lambda i: (0, 0)),
            pl.BlockSpec((64, 32), lambda i: (0, 0)),
            pl.BlockSpec((1, 32), lambda i: (0, 0)),
            pl.BlockSpec((32, 2), lambda i: (0, 0)),
            pl.BlockSpec((1, 2), lambda i: (0, 0)),
        ],
        out_specs=[
            pl.BlockSpec((BN, 2), lambda i: (i, 0)),
            pl.BlockSpec((1, 1), lambda i: (0, 0)),
        ],
        out_shape=[
            jax.ShapeDtypeStruct((N, 2), jnp.float32),
            jax.ShapeDtypeStruct((1, 1), jnp.float32),
        ],
        scratch_shapes=[pltpu.SMEM((2,), jnp.float32)],
    )(a0, a1, g2, d0, d1, b2r, fc1_W, fc1_br, fc2_W, fc2_br)


def kernel(x, edge_index, W1, b1, W2, b2, fc1_W, fc1_b, fc2_W, fc2_b):
    pad = EPAD - E
    src_p = jnp.concatenate(
        [edge_index[0], jnp.zeros((pad,), jnp.int32)]).reshape(NW, NCH, CHUNK)
    dst_p = jnp.concatenate(
        [edge_index[1], jnp.full((pad,), N, jnp.int32)]).reshape(NW, NCH, CHUNK)

    ones_c = jnp.ones((CHUNK,), jnp.float32)
    zeros_d = jnp.zeros((NPAD,), jnp.float32)
    zeros_64 = jnp.zeros((NPAD, D), jnp.float32)

    deg_parts = _deg_kernel(dst_p, ones_c, zeros_d)
    d0 = deg_parts[0, :N, None]
    d1 = deg_parts[1, :N, None]

    g1a, g1b = _mm_scale(x, W1, d0, d1)
    acc_a, acc_b = _scatter_dual(g1a, g1b, src_p, dst_p, zeros_64)
    g2 = _layer2(acc_a[0, :N], acc_a[1, :N], acc_b[0, :N], acc_b[1, :N],
                 g1a, g1b, d0, d1, b1[None, :], W2)
    acc2, = _scatter_single(g2, src_p, dst_p, zeros_64)
    assign, var = _head(acc2[0, :N], acc2[1, :N], g2, d0, d1, b2[None, :],
                        fc1_W, fc1_b[None, :], fc2_W, fc2_b[None, :])
    return assign, var[0, 0]


# speedup vs baseline: 11.6348x; 1.0022x over previous
import functools

import jax
import jax.numpy as jnp
from jax import lax
from jax.experimental import pallas as pl
from jax.experimental.pallas import tpu as pltpu
from jax.experimental.pallas import tpu_sc as plsc

N = 10000
E = 320000
NC = 2
NS = 16
NW = NC * NS
CHUNK = 128
NCH = 80
EPW = NCH * CHUNK
EPAD = NW * EPW
NPAD = 10240
SL = NPAD // NS
D = 64

_mesh = plsc.VectorSubcoreMesh(core_axis_name="c", subcore_axis_name="s")


@functools.partial(
    pl.kernel, mesh=_mesh,
    compiler_params=pltpu.CompilerParams(use_tc_tiling_on_sc=False),
    out_type=jax.ShapeDtypeStruct((NC, NPAD), jnp.float32),
    scratch_types=[
        pltpu.VMEM((NCH, CHUNK), jnp.int32),
        pltpu.VMEM((CHUNK,), jnp.float32),
        pltpu.VMEM_SHARED((NPAD,), jnp.float32),
    ],
)
def _deg_kernel(dst_hbm, ones_hbm, zeros_hbm, out_hbm, idx_v, ones_v, acc_sh):
    cid = lax.axis_index("c")
    sid = lax.axis_index("s")
    wid = sid * NC + cid
    pltpu.sync_copy(zeros_hbm.at[pl.ds(sid * SL, SL)], acc_sh.at[pl.ds(sid * SL, SL)])
    pltpu.sync_copy(ones_hbm, ones_v)
    pltpu.sync_copy(dst_hbm.at[wid], idx_v)
    plsc.subcore_barrier()

    def body(j, carry):
        pltpu.sync_copy(ones_v, acc_sh.at[idx_v.at[j]], add=True)
        return carry

    lax.fori_loop(0, NCH, body, 0)
    plsc.subcore_barrier()
    pltpu.sync_copy(acc_sh.at[pl.ds(sid * SL, SL)], out_hbm.at[cid, pl.ds(sid * SL, SL)])


U = 4


def _make_scatter(num_tables):
    out_struct = jax.ShapeDtypeStruct((NC, NPAD, D), jnp.float32)

    @functools.partial(
        pl.kernel, mesh=_mesh,
        compiler_params=pltpu.CompilerParams(use_tc_tiling_on_sc=False),
        out_type=[out_struct] * num_tables,
        scratch_types=[
            pltpu.VMEM((NCH, CHUNK), jnp.int32),
            pltpu.VMEM((NCH, CHUNK), jnp.int32),
        ] + [pltpu.VMEM((CHUNK, D), jnp.float32)] * U
          + [pltpu.VMEM_SHARED((NPAD, D), jnp.float32)]
          + [pltpu.SemaphoreType.DMA] * (2 * U),
    )
    def _scatter_kernel(*args):
        tables = args[:num_tables]
        src_hbm, dst_hbm, zeros_hbm = args[num_tables:num_tables + 3]
        outs = args[num_tables + 3:2 * num_tables + 3]
        rest = args[2 * num_tables + 3:]
        srcv, dstv = rest[0], rest[1]
        rows = rest[2:2 + U]
        acc_sh = rest[2 + U]
        gsem = rest[3 + U:3 + 2 * U]
        ssem = rest[3 + 2 * U:3 + 3 * U]
        cid = lax.axis_index("c")
        sid = lax.axis_index("s")
        wid = sid * NC + cid
        sl = pl.ds(sid * SL, SL)
        pltpu.sync_copy(src_hbm.at[wid], srcv)
        pltpu.sync_copy(dst_hbm.at[wid], dstv)
        for g_hbm, out_hbm in zip(tables, outs):
            pltpu.sync_copy(zeros_hbm.at[sl], acc_sh.at[sl])
            plsc.subcore_barrier()

            for b in range(U):
                pltpu.async_copy(g_hbm.at[srcv.at[b]], rows[b], gsem[b])

            def round_body(r, carry):
                base = U * r
                for b in range(U):
                    pltpu.make_async_copy(
                        g_hbm.at[srcv.at[base + b]], rows[b], gsem[b]).wait()
                    pltpu.async_copy(
                        rows[b], acc_sh.at[dstv.at[base + b]], ssem[b], add=True)
                for b in range(U):
                    pltpu.make_async_copy(
                        rows[b], acc_sh.at[dstv.at[base + b]], ssem[b]).wait()
                    pltpu.async_copy(
                        g_hbm.at[srcv.at[base + U + b]], rows[b], gsem[b])
                return carry

            lax.fori_loop(0, NCH // U - 1, round_body, 0)

            base = NCH - U
            for b in range(U):
                pltpu.make_async_copy(
                    g_hbm.at[srcv.at[base + b]], rows[b], gsem[b]).wait()
                pltpu.async_copy(
                    rows[b], acc_sh.at[dstv.at[base + b]], ssem[b], add=True)
            for b in range(U):
                pltpu.make_async_copy(
                    rows[b], acc_sh.at[dstv.at[base + b]], ssem[b]).wait()

            plsc.subcore_barrier()
            pltpu.sync_copy(acc_sh.at[sl], out_hbm.at[cid, sl])

    return _scatter_kernel


_scatter_dual = _make_scatter(2)
_scatter_single = _make_scatter(1)


BN = 1000


def _mm_scale_body(x_ref, w_ref, d0_ref, d1_ref, outa_ref, outb_ref):
    dinv = lax.rsqrt(d0_ref[...] + d1_ref[...] + 1.0)
    h = jnp.dot(x_ref[...], w_ref[...], preferred_element_type=jnp.float32)
    g = h * dinv
    outa_ref[...] = g[:, :D]
    outb_ref[...] = g[:, D:]


def _mm_scale(x, W, d0, d1):
    return pl.pallas_call(
        _mm_scale_body,
        grid=(N // BN,),
        in_specs=[
            pl.BlockSpec((BN, 128), lambda i: (i, 0)),
            pl.BlockSpec((128, 128), lambda i: (0, 0)),
            pl.BlockSpec((BN, 1), lambda i: (i, 0)),
            pl.BlockSpec((BN, 1), lambda i: (i, 0)),
        ],
        out_specs=[
            pl.BlockSpec((BN, D), lambda i: (i, 0)),
            pl.BlockSpec((BN, D), lambda i: (i, 0)),
        ],
        out_shape=[
            jax.ShapeDtypeStruct((N, D), jnp.float32),
            jax.ShapeDtypeStruct((N, D), jnp.float32),
        ],
    )(x, W, d0, d1)


def _layer2_body(aa0_ref, aa1_ref, ab0_ref, ab1_ref, ga_ref, gb_ref,
                 d0_ref, d1_ref, b1_ref, w2_ref, out_ref):
    dinv = lax.rsqrt(d0_ref[...] + d1_ref[...] + 1.0)
    b1 = b1_ref[...]
    h1a = jnp.maximum(dinv * (aa0_ref[...] + aa1_ref[...] + ga_ref[...]) + b1[:, :D], 0.0)
    h1b = jnp.maximum(dinv * (ab0_ref[...] + ab1_ref[...] + gb_ref[...]) + b1[:, D:], 0.0)
    w2 = w2_ref[...]
    h2 = (jnp.dot(h1a, w2[:D], preferred_element_type=jnp.float32)
          + jnp.dot(h1b, w2[D:], preferred_element_type=jnp.float32))
    out_ref[...] = dinv * h2


def _layer2(aa0, aa1, ab0, ab1, ga, gb, d0, d1, b1r, W2):
    return pl.pallas_call(
        _layer2_body,
        grid=(N // BN,),
        in_specs=[
            pl.BlockSpec((BN, D), lambda i: (i, 0)),
            pl.BlockSpec((BN, D), lambda i: (i, 0)),
            pl.BlockSpec((BN, D), lambda i: (i, 0)),
            pl.BlockSpec((BN, D), lambda i: (i, 0)),
            pl.BlockSpec((BN, D), lambda i: (i, 0)),
            pl.BlockSpec((BN, D), lambda i: (i, 0)),
            pl.BlockSpec((BN, 1), lambda i: (i, 0)),
            pl.BlockSpec((BN, 1), lambda i: (i, 0)),
            pl.BlockSpec((1, 128), lambda i: (0, 0)),
            pl.BlockSpec((128, 64), lambda i: (0, 0)),
        ],
        out_specs=pl.BlockSpec((BN, 64), lambda i: (i, 0)),
        out_shape=jax.ShapeDtypeStruct((N, 64), jnp.float32),
    )(aa0, aa1, ab0, ab1, ga, gb, d0, d1, b1r, W2)


_M = float(N * 2)


def _head_body(a0_ref, a1_ref, g2_ref, d0_ref, d1_ref, b2_ref,
               fc1w_ref, fc1b_ref, fc2w_ref, fc2b_ref,
               assign_ref, var_ref, acc_ref):
    i = pl.program_id(0)

    @pl.when(i == 0)
    def _():
        acc_ref[0] = 0.0
        acc_ref[1] = 0.0

    dinv = lax.rsqrt(d0_ref[...] + d1_ref[...] + 1.0)
    h2 = dinv * (a0_ref[...] + a1_ref[...] + g2_ref[...]) + b2_ref[...]
    a1 = jnp.tanh(jnp.dot(h2, fc1w_ref[...], preferred_element_type=jnp.float32)
                  + fc1b_ref[...])
    logits = jnp.dot(a1, fc2w_ref[...], preferred_element_type=jnp.float32) + fc2b_ref[...]
    m = jnp.max(logits, axis=1, keepdims=True)
    e = jnp.exp(logits - m)
    assign = e / jnp.sum(e, axis=1, keepdims=True)
    assign_ref[...] = assign
    c = assign - 0.5
    acc_ref[0] += jnp.sum(c)
    acc_ref[1] += jnp.sum(c * c)

    @pl.when(i == pl.num_programs(0) - 1)
    def _():
        s = acc_ref[0]
        q = acc_ref[1]
        v = (q - s * s / _M) / (_M - 1.0)
        var_ref[...] = jnp.broadcast_to(v, (1, 1))


def _head(a0, a1, g2, d0, d1, b2r, fc1_W, fc1_br, fc2_W, fc2_br):
    return pl.pallas_call(
        _head_body,
        grid=(N // BN,),
        in_specs=[
            pl.BlockSpec((BN, 64), lambda i: (i, 0)),
            pl.BlockSpec((BN, 64), lambda i: (i, 0)),
            pl.BlockSpec((BN, 64), lambda i: (i, 0)),
            pl.BlockSpec((BN, 1), lambda i: (i, 0)),
            pl.BlockSpec((BN, 1), lambda i: (i, 0)),
            pl.BlockSpec((1, 64), lambda i: (0, 0)),
            pl.BlockSpec((64, 32), lambda i: (0, 0)),
            pl.BlockSpec((1, 32), lambda i: (0, 0)),
            pl.BlockSpec((32, 2), lambda i: (0, 0)),
            pl.BlockSpec((1, 2), lambda i: (0, 0)),
        ],
        out_specs=[
            pl.BlockSpec((BN, 2), lambda i: (i, 0)),
            pl.BlockSpec((1, 1), lambda i: (0, 0)),
        ],
        out_shape=[
            jax.ShapeDtypeStruct((N, 2), jnp.float32),
            jax.ShapeDtypeStruct((1, 1), jnp.float32),
        ],
        scratch_shapes=[pltpu.SMEM((2,), jnp.float32)],
    )(a0, a1, g2, d0, d1, b2r, fc1_W, fc1_br, fc2_W, fc2_br)


def kernel(x, edge_index, W1, b1, W2, b2, fc1_W, fc1_b, fc2_W, fc2_b):
    pad = EPAD - E
    src_p = jnp.concatenate(
        [edge_index[0], jnp.zeros((pad,), jnp.int32)]).reshape(NW, NCH, CHUNK)
    pad_dst = N + (jnp.arange(pad, dtype=jnp.int32) % (NPAD - N))
    dst_p = jnp.concatenate([edge_index[1], pad_dst]).reshape(NW, NCH, CHUNK)

    ones_c = jnp.ones((CHUNK,), jnp.float32)
    zeros_d = jnp.zeros((NPAD,), jnp.float32)
    zeros_64 = jnp.zeros((NPAD, D), jnp.float32)

    deg_parts = _deg_kernel(dst_p, ones_c, zeros_d)
    d0 = deg_parts[0, :N, None]
    d1 = deg_parts[1, :N, None]

    g1a, g1b = _mm_scale(x, W1, d0, d1)
    acc_a, acc_b = _scatter_dual(g1a, g1b, src_p, dst_p, zeros_64)
    g2 = _layer2(acc_a[0, :N], acc_a[1, :N], acc_b[0, :N], acc_b[1, :N],
                 g1a, g1b, d0, d1, b1[None, :], W2)
    acc2, = _scatter_single(g2, src_p, dst_p, zeros_64)
    assign, var = _head(acc2[0, :N], acc2[1, :N], g2, d0, d1, b2[None, :],
                        fc1_W, fc1_b[None, :], fc2_W, fc2_b[None, :])
    return assign, var[0, 0]
